# baseline (speedup 1.0000x reference)
import functools

import jax
import jax.numpy as jnp
from jax import lax
from jax.experimental import pallas as pl
from jax.experimental.pallas import tpu as pltpu
from jax.experimental.pallas import tpu_sc as plsc

_MAX_LEN = 512
_VOCAB = 32000
_UNK = _VOCAB + 1
_CLS = _VOCAB + 2
_SEP = _VOCAB + 3

_NC = 2
_NS = 16
_NW = _NC * _NS

_RB = 16
_IN_OFF = 16
_NVREG = _MAX_LEN // 16


@functools.cache
def _tokens_call(B, L):
    assert B % (_NW * _RB) == 0
    rpw = B // _NW
    nb = rpw // _RB
    in_w = _IN_OFF + _RB * L + 16
    out_w = _RB * _MAX_LEN

    mesh = plsc.VectorSubcoreMesh(
        core_axis_name="c", subcore_axis_name="s", num_cores=_NC, num_subcores=_NS
    )

    @functools.partial(
        pl.kernel,
        out_type=jax.ShapeDtypeStruct((B * _MAX_LEN,), jnp.int32),
        mesh=mesh,
        scratch_types=[
            pltpu.VMEM((rpw,), jnp.int32),
            pltpu.VMEM((2 * in_w,), jnp.int32),
            pltpu.VMEM((2 * out_w,), jnp.int32),
            pltpu.SemaphoreType.DMA,
            pltpu.SemaphoreType.DMA,
        ],
        compiler_params=pltpu.CompilerParams(
            use_tc_tiling_on_sc=False, needs_layout_passes=False
        ),
    )
    def tokens_kernel(pieces_hbm, lengths_hbm, tokens_hbm, lens_v, inbuf, outbuf,
                      sem_in, sem_out):
        wid = lax.axis_index("s") * _NC + lax.axis_index("c")
        base = wid * rpw
        pltpu.sync_copy(lengths_hbm.at[pl.ds(base, rpw)], lens_v)
        lane = lax.iota(jnp.int32, 16)
        fix_val = jnp.where(lane == 0, jnp.int32(_CLS), jnp.int32(_SEP))
        fix_mask = lane < 2
        zeros16 = jnp.zeros((16,), jnp.int32)

        def start_in(b, t):
            pltpu.async_copy(
                pieces_hbm.at[pl.ds((base + b * _RB) * L, _RB * L)],
                inbuf.at[pl.ds(t * in_w + _IN_OFF, _RB * L)],
                sem_in,
            )

        def wait_in():
            pltpu.make_async_copy(
                pieces_hbm.at[pl.ds(0, _RB * L)],
                inbuf.at[pl.ds(_IN_OFF, _RB * L)],
                sem_in,
            ).wait()

        def start_out(b, t):
            pltpu.async_copy(
                outbuf.at[pl.ds(t * out_w, out_w)],
                tokens_hbm.at[pl.ds((base + b * _RB) * _MAX_LEN, out_w)],
                sem_out,
            )

        def wait_out():
            pltpu.make_async_copy(
                outbuf.at[pl.ds(0, out_w)],
                tokens_hbm.at[pl.ds(0, out_w)],
                sem_out,
            ).wait()

        start_in(0, 0)

        def batch(b, carry):
            t = lax.rem(b, 2)
            wait_in()

            @pl.when(b + 1 < nb)
            def _():
                start_in(b + 1, 1 - t)

            @pl.when(b >= 2)
            def _():
                wait_out()

            lens_vec = lens_v[pl.ds(b * _RB, _RB)]
            ibase0 = t * in_w + _IN_OFF
            obase0 = t * out_w
            for rr in range(_RB):
                ln = lens_vec[rr]
                rbase = ibase0 + rr * L - 1
                obase = obase0 + rr * _MAX_LEN
                nv = lax.shift_right_logical(ln + 1, 4) + 1
                ng = lax.shift_right_logical(nv + 3, 2)

                def vfull4(g, c, rbase=rbase, obase=obase, ln=ln):
                    for u in range(4):
                        start = g * 64 + u * 16
                        v = inbuf[pl.ds(rbase + start, 16)]
                        v = jnp.where(v == 0, _UNK, v)
                        col = start + lane
                        outbuf[pl.ds(obase + start, 16)] = jnp.where(
                            col <= ln, v, 0
                        )
                    return c

                def vzero4(g, c, obase=obase):
                    for u in range(4):
                        outbuf[pl.ds(obase + g * 64 + u * 16, 16)] = zeros16
                    return c

                lax.fori_loop(0, ng, vfull4, 0)
                lax.fori_loop(ng, _NVREG // 4, vzero4, 0)
                fix_idx = obase + jnp.where(lane == 0, 0, ln + 1)
                plsc.store_scatter(outbuf, [fix_idx], fix_val, mask=fix_mask)
            start_out(b, t)
            return carry

        lax.fori_loop(0, nb, batch, 0)
        wait_out()
        wait_out()

    return tokens_kernel


def kernel(pieces, lengths):
    B, L = pieces.shape
    tokens = _tokens_call(B, L)(pieces.reshape(-1), lengths.astype(jnp.int32))
    segments = jnp.zeros((B, _MAX_LEN), jnp.int32)
    return tokens.reshape(B, _MAX_LEN), segments


# speedup vs baseline: 1.3201x; 1.3201x over previous
import functools

import jax
import jax.numpy as jnp
from jax import lax
from jax.experimental import pallas as pl
from jax.experimental.pallas import tpu as pltpu
from jax.experimental.pallas import tpu_sc as plsc

_MAX_LEN = 512
_VOCAB = 32000
_UNK = _VOCAB + 1
_CLS = _VOCAB + 2
_SEP = _VOCAB + 3

_NC = 2
_NS = 16
_NW = _NC * _NS

_RB = 16
_IN_OFF = 16
_NVREG = _MAX_LEN // 16


@functools.cache
def _tokens_call(B, L):
    assert B % (_NW * _RB) == 0
    rpw = B // _NW
    nb = rpw // _RB
    in_w = _IN_OFF + _RB * L + 16
    out_w = _RB * _MAX_LEN

    mesh = plsc.VectorSubcoreMesh(
        core_axis_name="c", subcore_axis_name="s", num_cores=_NC, num_subcores=_NS
    )

    @functools.partial(
        pl.kernel,
        out_type=jax.ShapeDtypeStruct((B * _MAX_LEN,), jnp.int32),
        mesh=mesh,
        scratch_types=[
            pltpu.VMEM((rpw,), jnp.int32),
            pltpu.VMEM((2 * in_w,), jnp.int32),
            pltpu.VMEM((2 * out_w,), jnp.int32),
            pltpu.SemaphoreType.DMA,
            pltpu.SemaphoreType.DMA,
        ],
        compiler_params=pltpu.CompilerParams(
            use_tc_tiling_on_sc=False, needs_layout_passes=False
        ),
    )
    def tokens_kernel(pieces_hbm, lengths_hbm, tokens_hbm, lens_v, inbuf, outbuf,
                      sem_in, sem_out):
        wid = lax.axis_index("s") * _NC + lax.axis_index("c")
        base = wid * rpw
        pltpu.sync_copy(lengths_hbm.at[pl.ds(base, rpw)], lens_v)
        lane = lax.iota(jnp.int32, 16)
        fix_val = jnp.where(lane == 0, jnp.int32(_CLS), jnp.int32(_SEP))
        fix_mask = lane < 2

        def start_in(b, t):
            pltpu.async_copy(
                pieces_hbm.at[pl.ds((base + b * _RB) * L, _RB * L)],
                inbuf.at[pl.ds(t * in_w + _IN_OFF, _RB * L)],
                sem_in,
            )

        def wait_in():
            pltpu.make_async_copy(
                pieces_hbm.at[pl.ds(0, _RB * L)],
                inbuf.at[pl.ds(_IN_OFF, _RB * L)],
                sem_in,
            ).wait()

        def start_out(b, t):
            pltpu.async_copy(
                outbuf.at[pl.ds(t * out_w, out_w)],
                tokens_hbm.at[pl.ds((base + b * _RB) * _MAX_LEN, out_w)],
                sem_out,
            )

        def wait_out():
            pltpu.make_async_copy(
                outbuf.at[pl.ds(0, out_w)],
                tokens_hbm.at[pl.ds(0, out_w)],
                sem_out,
            ).wait()

        start_in(0, 0)

        def batch(b, carry):
            t = lax.rem(b, 2)
            wait_in()

            @pl.when(b + 1 < nb)
            def _():
                start_in(b + 1, 1 - t)

            @pl.when(b >= 2)
            def _():
                wait_out()

            lens_vec = lens_v[pl.ds(b * _RB, _RB)]
            ibase0 = t * in_w + _IN_OFF
            obase0 = t * out_w
            for rr in range(_RB):
                ln = lens_vec[rr]
                rbase = ibase0 + rr * L - 1
                obase = obase0 + rr * _MAX_LEN

                @plsc.parallel_loop(0, _NVREG, unroll=4)
                def _(j, rbase=rbase, obase=obase, ln=ln):
                    start = j * 16
                    v = inbuf[pl.ds(rbase + start, 16)]
                    v = jnp.where(v == 0, _UNK, v)
                    col = start + lane
                    outbuf[pl.ds(obase + start, 16)] = jnp.where(col <= ln, v, 0)
                fix_idx = obase + jnp.where(lane == 0, 0, ln + 1)
                plsc.store_scatter(outbuf, [fix_idx], fix_val, mask=fix_mask)
            start_out(b, t)
            return carry

        lax.fori_loop(0, nb, batch, 0)
        wait_out()
        wait_out()

    return tokens_kernel


def kernel(pieces, lengths):
    B, L = pieces.shape
    tokens = _tokens_call(B, L)(pieces.reshape(-1), lengths.astype(jnp.int32))
    segments = jnp.zeros((B, _MAX_LEN), jnp.int32)
    return tokens.reshape(B, _MAX_LEN), segments
